# baseline (speedup 1.0000x reference)
import jax
import jax.numpy as jnp
from jax.experimental import pallas as pl
from jax.experimental.pallas import tpu as pltpu
from jax.experimental.pallas import tpu_sc as plsc

ROWS = 16


def kernel(x, weight):
    batch, fields = x.shape
    dim = weight.shape[1]
    idx = x.astype(jnp.int32)

    mesh = plsc.VectorSubcoreMesh(core_axis_name="core", subcore_axis_name="subcore")

    @pl.kernel(
        out_type=jax.ShapeDtypeStruct((batch, fields, dim), weight.dtype),
        mesh=mesh,
        scratch_types=[pltpu.SemaphoreType.DMA],
        compiler_params=pltpu.CompilerParams(use_tc_tiling_on_sc=False),
    )
    def gather_kernel(w_hbm, i_hbm, o_hbm, sem):
        def body(i_vmem, o_vmem):
            @pl.loop(0, ROWS)
            def _(r):
                pltpu.async_copy(w_hbm.at[i_vmem.at[r]], o_vmem.at[r], sem)

            pltpu.make_async_copy(o_hbm.at[pl.ds(0, ROWS)], o_vmem, sem).wait()

        pltpu.emit_pipeline(
            body,
            grid=(batch // ROWS,),
            in_specs=[pl.BlockSpec((ROWS, fields), index_map=lambda i: (i, 0))],
            out_specs=[pl.BlockSpec((ROWS, fields, dim), index_map=lambda i: (i, 0, 0))],
            core_axis_name=("core", "subcore"),
            dimension_semantics=(pltpu.PARALLEL,),
        )(i_hbm, o_hbm)

    return gather_kernel(weight, idx)
